# Initial kernel scaffold; baseline (speedup 1.0000x reference)
import functools

import jax
import jax.numpy as jnp
from jax import lax
from jax.experimental import pallas as pl
from jax.experimental.pallas import tpu as pltpu
from jax.experimental.pallas import tpu_sc as plsc

N0 = 5000
N1 = 5000
N = N0 + N1
E = 320000
ER = E // 2
Q = 100000
IN = 128
HID = 128
OUT = 64

NC = 2
NS = 16
NW = NC * NS
K = 80
CE = ER // NS
NTRIPS = CE // K
RPT = N // NS
ZROWS = 125
CNT_CHUNK = 640
QT = 3200

_f32 = jnp.float32


def _m1_body(x0_ref, x1_ref, lw0_ref, lb0_ref, lw1_ref, lb1_ref, wrel_ref,
             wroot_ref, b1_ref, t0_ref, t1_ref, root_ref):
    h0 = jnp.dot(x0_ref[...], lw0_ref[...], preferred_element_type=_f32) + lb0_ref[...]
    h1 = jnp.dot(x1_ref[...], lw1_ref[...], preferred_element_type=_f32) + lb1_ref[...]
    h = jnp.concatenate([h0, h1], axis=0)
    t0_ref[...] = jnp.dot(h, wrel_ref[0], preferred_element_type=_f32)
    t1_ref[...] = jnp.dot(h, wrel_ref[1], preferred_element_type=_f32)
    root_ref[...] = jnp.dot(h, wroot_ref[...], preferred_element_type=_f32) + b1_ref[...]


_m1 = pl.pallas_call(
    _m1_body,
    out_shape=[
        jax.ShapeDtypeStruct((N, HID), _f32),
        jax.ShapeDtypeStruct((N, HID), _f32),
        jax.ShapeDtypeStruct((N, HID), _f32),
    ],
)


def _m2_body(root_ref, a0_ref, a1_ref, c0_ref, c1_ref, wrel_ref, wroot_ref,
             b_ref, t0_ref, t1_ref, root2_ref):
    inv0 = 1.0 / jnp.maximum(c0_ref[...], 1.0)
    inv1 = 1.0 / jnp.maximum(c1_ref[...], 1.0)
    z = root_ref[...] + a0_ref[...] * inv0[:, None] + a1_ref[...] * inv1[:, None]
    z = jnp.maximum(z, 0.0)
    t0_ref[...] = jnp.dot(z, wrel_ref[0], preferred_element_type=_f32)
    t1_ref[...] = jnp.dot(z, wrel_ref[1], preferred_element_type=_f32)
    root2_ref[...] = jnp.dot(z, wroot_ref[...], preferred_element_type=_f32) + b_ref[...]


_m2 = pl.pallas_call(
    _m2_body,
    out_shape=[
        jax.ShapeDtypeStruct((N, OUT), _f32),
        jax.ShapeDtypeStruct((N, OUT), _f32),
        jax.ShapeDtypeStruct((N, OUT), _f32),
    ],
)


def _m3_body(root_ref, a0_ref, a1_ref, c0_ref, c1_ref, wuv_ref, buv_ref, uv_ref):
    inv0 = 1.0 / jnp.maximum(c0_ref[...], 1.0)
    inv1 = 1.0 / jnp.maximum(c1_ref[...], 1.0)
    z = root_ref[...] + a0_ref[...] * inv0[:, None] + a1_ref[...] * inv1[:, None]
    uv_ref[...] = jnp.dot(z, wuv_ref[...], preferred_element_type=_f32) + buv_ref[...]


_m3 = pl.pallas_call(
    _m3_body,
    out_shape=jax.ShapeDtypeStruct((N, 2), _f32),
)


def _zero_rows(ref, rows, d):
    def row_body(r, carry):
        def col_body(j, carry2):
            ref[r, pl.ds(j * 16, 16)] = jnp.zeros((16,), _f32)
            return carry2
        return lax.fori_loop(0, d // 16, col_body, carry)
    lax.fori_loop(0, rows, row_body, 0)


def _fill_vec(ref, n, value):
    def body(j, carry):
        ref[pl.ds(j * 16, 16)] = jnp.full((16,), value, _f32)
        return carry
    lax.fori_loop(0, n // 16, body, 0)


def _make_agg(d, with_counts):
    mesh = plsc.VectorSubcoreMesh(
        core_axis_name="c", subcore_axis_name="s", num_cores=NC, num_subcores=NS)
    out_type = [
        jax.ShapeDtypeStruct((N, d), _f32),
        jax.ShapeDtypeStruct((N, d), _f32),
    ]
    scratch = [
        pltpu.VMEM((K,), jnp.int32),
        pltpu.VMEM((K,), jnp.int32),
        pltpu.VMEM((K, d), _f32),
        pltpu.VMEM((ZROWS, d), _f32),
        pltpu.VMEM_SHARED((N, d), _f32),
    ]
    if with_counts:
        out_type += [
            jax.ShapeDtypeStruct((N,), _f32),
            jax.ShapeDtypeStruct((N,), _f32),
        ]
        scratch += [
            pltpu.VMEM((K,), _f32),
            pltpu.VMEM((CNT_CHUNK,), _f32),
            pltpu.VMEM_SHARED((N,), _f32),
        ]

    def body(t0_hbm, t1_hbm, src_hbm, dst_hbm, agg0_out, agg1_out, *rest):
        if with_counts:
            (cnt0_out, cnt1_out, sidx_v, didx_v, rows_v, zrows_v, acc_sh,
             ones_v, zcnt_v, cnt_sh) = rest
        else:
            sidx_v, didx_v, rows_v, zrows_v, acc_sh = rest
        c = lax.axis_index("c")
        s = lax.axis_index("s")

        _zero_rows(zrows_v, ZROWS, d)
        for kk in range(RPT // ZROWS):
            pltpu.sync_copy(zrows_v, acc_sh.at[pl.ds(s * RPT + kk * ZROWS, ZROWS)])
        if with_counts:
            _fill_vec(ones_v, K, 1.0)
            _fill_vec(zcnt_v, CNT_CHUNK, 0.0)

            @pl.when(s < NS - 1)
            def _():
                pltpu.sync_copy(zcnt_v, cnt_sh.at[pl.ds(s * CNT_CHUNK, CNT_CHUNK)])

            @pl.when(s == NS - 1)
            def _():
                pltpu.sync_copy(zcnt_v.at[pl.ds(0, N - (NS - 1) * CNT_CHUNK)],
                                cnt_sh.at[pl.ds((NS - 1) * CNT_CHUNK,
                                                N - (NS - 1) * CNT_CHUNK)])
        plsc.subcore_barrier()

        base_e = c * ER + s * CE

        def step(i, carry):
            off = pl.multiple_of(base_e + i * K, 8)
            pltpu.sync_copy(src_hbm.at[pl.ds(off, K)], sidx_v)
            pltpu.sync_copy(dst_hbm.at[pl.ds(off, K)], didx_v)

            @pl.when(c == 0)
            def _():
                pltpu.sync_copy(t0_hbm.at[sidx_v], rows_v)

            @pl.when(c == 1)
            def _():
                pltpu.sync_copy(t1_hbm.at[sidx_v], rows_v)

            pltpu.sync_copy(rows_v, acc_sh.at[didx_v], add=True)
            if with_counts:
                pltpu.sync_copy(ones_v, cnt_sh.at[didx_v], add=True)
            return carry

        lax.fori_loop(0, NTRIPS, step, 0)
        plsc.subcore_barrier()

        row0 = s * RPT

        @pl.when(c == 0)
        def _():
            pltpu.sync_copy(acc_sh.at[pl.ds(row0, RPT)], agg0_out.at[pl.ds(row0, RPT)])

        @pl.when(c == 1)
        def _():
            pltpu.sync_copy(acc_sh.at[pl.ds(row0, RPT)], agg1_out.at[pl.ds(row0, RPT)])

        if with_counts:
            tail = N - (NS - 1) * CNT_CHUNK

            @pl.when(jnp.logical_and(c == 0, s < NS - 1))
            def _():
                pltpu.sync_copy(cnt_sh.at[pl.ds(s * CNT_CHUNK, CNT_CHUNK)],
                                cnt0_out.at[pl.ds(s * CNT_CHUNK, CNT_CHUNK)])

            @pl.when(jnp.logical_and(c == 0, s == NS - 1))
            def _():
                pltpu.sync_copy(cnt_sh.at[pl.ds((NS - 1) * CNT_CHUNK, tail)],
                                cnt0_out.at[pl.ds((NS - 1) * CNT_CHUNK, tail)])

            @pl.when(jnp.logical_and(c == 1, s < NS - 1))
            def _():
                pltpu.sync_copy(cnt_sh.at[pl.ds(s * CNT_CHUNK, CNT_CHUNK)],
                                cnt1_out.at[pl.ds(s * CNT_CHUNK, CNT_CHUNK)])

            @pl.when(jnp.logical_and(c == 1, s == NS - 1))
            def _():
                pltpu.sync_copy(cnt_sh.at[pl.ds((NS - 1) * CNT_CHUNK, tail)],
                                cnt1_out.at[pl.ds((NS - 1) * CNT_CHUNK, tail)])

    return pl.kernel(body, out_type=out_type, mesh=mesh, scratch_types=scratch)


_agg_l1 = _make_agg(HID, with_counts=True)
_agg_l2 = _make_agg(OUT, with_counts=False)


def _make_decode():
    mesh = plsc.VectorSubcoreMesh(
        core_axis_name="c", subcore_axis_name="s", num_cores=NC, num_subcores=NS)
    out_type = jax.ShapeDtypeStruct((Q,), _f32)
    scratch = [
        pltpu.VMEM((K,), jnp.int32),
        pltpu.VMEM((K,), jnp.int32),
        pltpu.VMEM((K,), _f32),
        pltpu.VMEM((K,), _f32),
        pltpu.VMEM((K,), _f32),
    ]

    def body(u_hbm, v_hbm, i0_hbm, i1_hbm, out_hbm, i0_v, i1_v, a_v, b_v, r_v):
        c = lax.axis_index("c")
        s = lax.axis_index("s")
        w = s * NC + c
        base = w * QT
        trips = jnp.where(w < NW - 1, QT // K, (Q - (NW - 1) * QT) // K)

        def step(i, carry):
            off = pl.multiple_of(base + i * K, 8)
            pltpu.sync_copy(i0_hbm.at[pl.ds(off, K)], i0_v)
            pltpu.sync_copy(i1_hbm.at[pl.ds(off, K)], i1_v)
            pltpu.sync_copy(u_hbm.at[i0_v], a_v)
            pltpu.sync_copy(v_hbm.at[i1_v], b_v)
            for j in range(K // 16):
                x = a_v[pl.ds(j * 16, 16)] + b_v[pl.ds(j * 16, 16)]
                r_v[pl.ds(j * 16, 16)] = 1.0 / (1.0 + jnp.exp(-x))
            pltpu.sync_copy(r_v, out_hbm.at[pl.ds(off, K)])
            return carry

        lax.fori_loop(0, trips, step, 0)

    return pl.kernel(body, out_type=out_type, mesh=mesh, scratch_types=scratch)


_decode = _make_decode()


def kernel(x0, x1, edge_index, index, lin0_W, lin0_b, lin1_W, lin1_b,
           W1_rel, W1_root, b1, W2_rel, W2_root, b2, fc_W, fc_b):
    src = jnp.asarray(edge_index[0], jnp.int32)
    dst = jnp.asarray(edge_index[1], jnp.int32)
    i0 = jnp.asarray(index[0], jnp.int32)
    i1 = jnp.asarray(index[1], jnp.int32)

    t1_0, t1_1, root1 = _m1(
        x0, x1, lin0_W, lin0_b.reshape(1, IN), lin1_W, lin1_b.reshape(1, IN),
        W1_rel, W1_root, b1.reshape(1, HID))
    agg1_0, agg1_1, cnt0, cnt1 = _agg_l1(t1_0, t1_1, src, dst)
    t2_0, t2_1, root2 = _m2(
        root1, agg1_0, agg1_1, cnt0, cnt1, W2_rel, W2_root, b2.reshape(1, OUT))
    agg2_0, agg2_1 = _agg_l2(t2_0, t2_1, src, dst)

    wuv = jnp.concatenate([fc_W[:OUT], fc_W[OUT:]], axis=1)
    buv = jnp.concatenate([fc_b, jnp.zeros((1,), _f32)]).reshape(1, 2)
    uv = _m3(root2, agg2_0, agg2_1, cnt0, cnt1, wuv, buv)
    out = _decode(uv[:, 0], uv[:, 1], i0, i1)
    return out.reshape(Q, 1)


# speedup vs baseline: 7.4145x; 7.4145x over previous
import functools

import jax
import jax.numpy as jnp
from jax import lax
from jax.experimental import pallas as pl
from jax.experimental.pallas import tpu as pltpu
from jax.experimental.pallas import tpu_sc as plsc

N0 = 5000
N1 = 5000
N = N0 + N1
E = 320000
ER = E // 2
Q = 100000
IN = 128
HID = 128
OUT = 64

NC = 2
NS = 16
NW = NC * NS
K = 80
CE = ER // NS
NTRIPS = CE // K
CHUNK = 640
TAIL = N - (NS - 1) * CHUNK
ZROWS = 80
CNT_CHUNK = 640
QT = 3200

_f32 = jnp.float32


def _m1_body(x0_ref, x1_ref, lw0_ref, lb0_ref, lw1_ref, lb1_ref, wrel_ref,
             wroot_ref, b1_ref, t0_ref, t1_ref, root_ref):
    h0 = jnp.dot(x0_ref[...], lw0_ref[...], preferred_element_type=_f32) + lb0_ref[...]
    h1 = jnp.dot(x1_ref[...], lw1_ref[...], preferred_element_type=_f32) + lb1_ref[...]
    h = jnp.concatenate([h0, h1], axis=0)
    t0_ref[...] = jnp.dot(h, wrel_ref[0], preferred_element_type=_f32)
    t1_ref[...] = jnp.dot(h, wrel_ref[1], preferred_element_type=_f32)
    root_ref[...] = jnp.dot(h, wroot_ref[...], preferred_element_type=_f32) + b1_ref[...]


_m1 = pl.pallas_call(
    _m1_body,
    out_shape=[
        jax.ShapeDtypeStruct((N, HID), _f32),
        jax.ShapeDtypeStruct((N, HID), _f32),
        jax.ShapeDtypeStruct((N, HID), _f32),
    ],
)


def _m2_body(root_ref, a0_ref, a1_ref, c0_ref, c1_ref, wrel_ref, wroot_ref,
             b_ref, tp_ref, root2_ref):
    inv0 = 1.0 / jnp.maximum(c0_ref[...], 1.0)
    inv1 = 1.0 / jnp.maximum(c1_ref[...], 1.0)
    z = root_ref[...] + a0_ref[...] * inv0[:, None] + a1_ref[...] * inv1[:, None]
    z = jnp.maximum(z, 0.0)
    tp_ref[...] = jnp.concatenate(
        [jnp.dot(z, wrel_ref[0], preferred_element_type=_f32),
         jnp.dot(z, wrel_ref[1], preferred_element_type=_f32)], axis=1)
    root2_ref[...] = jnp.dot(z, wroot_ref[...], preferred_element_type=_f32) + b_ref[...]


_m2 = pl.pallas_call(
    _m2_body,
    out_shape=[
        jax.ShapeDtypeStruct((N, 2 * OUT), _f32),
        jax.ShapeDtypeStruct((N, OUT), _f32),
    ],
)


def _m3_body(root_ref, a0_ref, a1_ref, c0_ref, c1_ref, wuv_ref, buv_ref, uv_ref):
    inv0 = 1.0 / jnp.maximum(c0_ref[...], 1.0)
    inv1 = 1.0 / jnp.maximum(c1_ref[...], 1.0)
    a0 = a0_ref[...][:, :OUT]
    a1 = a1_ref[...][:, OUT:]
    z = root_ref[...] + a0 * inv0[:, None] + a1 * inv1[:, None]
    uv_ref[...] = jnp.dot(z, wuv_ref[...], preferred_element_type=_f32) + buv_ref[...]


_m3 = pl.pallas_call(
    _m3_body,
    out_shape=jax.ShapeDtypeStruct((N, 2), _f32),
)


def _zero_rows(ref, rows, d):
    def row_body(r, carry):
        def col_body(j, carry2):
            ref[r, pl.ds(j * 16, 16)] = jnp.zeros((16,), _f32)
            return carry2
        return lax.fori_loop(0, d // 16, col_body, carry)
    lax.fori_loop(0, rows, row_body, 0)


def _fill_vec(ref, n, value):
    def body(j, carry):
        ref[pl.ds(j * 16, 16)] = jnp.full((16,), value, _f32)
        return carry
    lax.fori_loop(0, n // 16, body, 0)


def _make_agg(d, with_counts):
    mesh = plsc.VectorSubcoreMesh(
        core_axis_name="c", subcore_axis_name="s", num_cores=NC, num_subcores=NS)
    out_type = [
        jax.ShapeDtypeStruct((N, d), _f32),
        jax.ShapeDtypeStruct((N, d), _f32),
    ]
    scratch = [
        pltpu.VMEM((K,), jnp.int32),
        pltpu.VMEM((K,), jnp.int32),
        pltpu.VMEM((K, d), _f32),
        pltpu.VMEM((ZROWS, d), _f32),
        pltpu.VMEM_SHARED((N, d), _f32),
    ]
    if with_counts:
        out_type += [
            jax.ShapeDtypeStruct((N,), _f32),
            jax.ShapeDtypeStruct((N,), _f32),
        ]
        scratch += [
            pltpu.VMEM((K,), _f32),
            pltpu.VMEM((CNT_CHUNK,), _f32),
            pltpu.VMEM_SHARED((N,), _f32),
        ]

    def body(t0_hbm, t1_hbm, src_hbm, dst_hbm, agg0_out, agg1_out, *rest):
        if with_counts:
            (cnt0_out, cnt1_out, sidx_v, didx_v, rows_v, zrows_v, acc_sh,
             ones_v, zcnt_v, cnt_sh) = rest
        else:
            sidx_v, didx_v, rows_v, zrows_v, acc_sh = rest
        c = lax.axis_index("c")
        s = lax.axis_index("s")

        _zero_rows(zrows_v, ZROWS, d)

        @pl.when(s < NS - 1)
        def _():
            for kk in range(CHUNK // ZROWS):
                pltpu.sync_copy(zrows_v, acc_sh.at[pl.ds(s * CHUNK + kk * ZROWS, ZROWS)])

        @pl.when(s == NS - 1)
        def _():
            for kk in range(TAIL // ZROWS):
                pltpu.sync_copy(zrows_v, acc_sh.at[pl.ds((NS - 1) * CHUNK + kk * ZROWS, ZROWS)])
        if with_counts:
            _fill_vec(ones_v, K, 1.0)
            _fill_vec(zcnt_v, CNT_CHUNK, 0.0)

            @pl.when(s < NS - 1)
            def _():
                pltpu.sync_copy(zcnt_v, cnt_sh.at[pl.ds(s * CNT_CHUNK, CNT_CHUNK)])

            @pl.when(s == NS - 1)
            def _():
                pltpu.sync_copy(zcnt_v.at[pl.ds(0, N - (NS - 1) * CNT_CHUNK)],
                                cnt_sh.at[pl.ds((NS - 1) * CNT_CHUNK,
                                                N - (NS - 1) * CNT_CHUNK)])
        plsc.subcore_barrier()

        base_e = c * ER + s * CE

        def step(i, carry):
            off = pl.multiple_of(base_e + i * K, 8)
            pltpu.sync_copy(src_hbm.at[pl.ds(off, K)], sidx_v)
            pltpu.sync_copy(dst_hbm.at[pl.ds(off, K)], didx_v)

            @pl.when(c == 0)
            def _():
                pltpu.sync_copy(t0_hbm.at[sidx_v], rows_v)

            @pl.when(c == 1)
            def _():
                pltpu.sync_copy(t1_hbm.at[sidx_v], rows_v)

            pltpu.sync_copy(rows_v, acc_sh.at[didx_v], add=True)
            if with_counts:
                pltpu.sync_copy(ones_v, cnt_sh.at[didx_v], add=True)
            return carry

        lax.fori_loop(0, NTRIPS, step, 0)
        plsc.subcore_barrier()

        for cc, agg_out in ((0, agg0_out), (1, agg1_out)):
            @pl.when(jnp.logical_and(c == cc, s < NS - 1))
            def _(agg_out=agg_out):
                pltpu.sync_copy(acc_sh.at[pl.ds(s * CHUNK, CHUNK)],
                                agg_out.at[pl.ds(s * CHUNK, CHUNK)])

            @pl.when(jnp.logical_and(c == cc, s == NS - 1))
            def _(agg_out=agg_out):
                pltpu.sync_copy(acc_sh.at[pl.ds((NS - 1) * CHUNK, TAIL)],
                                agg_out.at[pl.ds((NS - 1) * CHUNK, TAIL)])

        if with_counts:
            tail = N - (NS - 1) * CNT_CHUNK
            for cc, cnt_out in ((0, cnt0_out), (1, cnt1_out)):
                @pl.when(jnp.logical_and(c == cc, s < NS - 1))
                def _(cnt_out=cnt_out):
                    pltpu.sync_copy(cnt_sh.at[pl.ds(s * CNT_CHUNK, CNT_CHUNK)], zcnt_v)
                    pltpu.sync_copy(zcnt_v, cnt_out.at[pl.ds(s * CNT_CHUNK, CNT_CHUNK)])

                @pl.when(jnp.logical_and(c == cc, s == NS - 1))
                def _(cnt_out=cnt_out):
                    pltpu.sync_copy(cnt_sh.at[pl.ds((NS - 1) * CNT_CHUNK, tail)],
                                    zcnt_v.at[pl.ds(0, tail)])
                    pltpu.sync_copy(zcnt_v.at[pl.ds(0, tail)],
                                    cnt_out.at[pl.ds((NS - 1) * CNT_CHUNK, tail)])

    return pl.kernel(body, out_type=out_type, mesh=mesh, scratch_types=scratch)


_agg_cache = functools.lru_cache(maxsize=None)(_make_agg)


def _make_decode():
    mesh = plsc.VectorSubcoreMesh(
        core_axis_name="c", subcore_axis_name="s", num_cores=NC, num_subcores=NS)
    out_type = jax.ShapeDtypeStruct((Q,), _f32)
    scratch = [
        pltpu.VMEM((N,), _f32),
        pltpu.VMEM((N,), _f32),
        pltpu.VMEM((K,), jnp.int32),
        pltpu.VMEM((K,), jnp.int32),
        pltpu.VMEM((K,), _f32),
    ]

    def body(u_hbm, v_hbm, i0_hbm, i1_hbm, out_hbm, u_v, v_v, i0_v, i1_v, r_v):
        c = lax.axis_index("c")
        s = lax.axis_index("s")
        w = s * NC + c
        base = w * QT
        trips = jnp.where(w < NW - 1, QT // K, (Q - (NW - 1) * QT) // K)
        pltpu.sync_copy(u_hbm, u_v)
        pltpu.sync_copy(v_hbm, v_v)

        def step(i, carry):
            off = pl.multiple_of(base + i * K, 8)
            pltpu.sync_copy(i0_hbm.at[pl.ds(off, K)], i0_v)
            pltpu.sync_copy(i1_hbm.at[pl.ds(off, K)], i1_v)
            for j in range(K // 16):
                a = plsc.load_gather(u_v, [i0_v[pl.ds(j * 16, 16)]])
                b = plsc.load_gather(v_v, [i1_v[pl.ds(j * 16, 16)]])
                x = a + b
                r_v[pl.ds(j * 16, 16)] = 1.0 / (1.0 + jnp.exp(-x))
            pltpu.sync_copy(r_v, out_hbm.at[pl.ds(off, K)])
            return carry

        lax.fori_loop(0, trips, step, 0)

    return pl.kernel(body, out_type=out_type, mesh=mesh, scratch_types=scratch,
                     compiler_params=pltpu.CompilerParams(
                         use_tc_tiling_on_sc=False, needs_layout_passes=False))


_decode_cache = functools.lru_cache(maxsize=None)(_make_decode)


def kernel(x0, x1, edge_index, index, lin0_W, lin0_b, lin1_W, lin1_b,
           W1_rel, W1_root, b1, W2_rel, W2_root, b2, fc_W, fc_b):
    src = jnp.asarray(edge_index[0], jnp.int32)
    dst = jnp.asarray(edge_index[1], jnp.int32)
    i0 = jnp.asarray(index[0], jnp.int32)
    i1 = jnp.asarray(index[1], jnp.int32)

    t1_0, t1_1, root1 = _m1(
        x0, x1, lin0_W, lin0_b.reshape(1, IN), lin1_W, lin1_b.reshape(1, IN),
        W1_rel, W1_root, b1.reshape(1, HID))
    agg1_0, agg1_1, cnt0, cnt1 = _agg_cache(HID, True)(t1_0, t1_1, src, dst)
    t2p, root2 = _m2(
        root1, agg1_0, agg1_1, cnt0, cnt1, W2_rel, W2_root, b2.reshape(1, OUT))
    agg2_0, agg2_1 = _agg_cache(2 * OUT, False)(t2p, t2p, src, dst)

    wuv = jnp.concatenate([fc_W[:OUT], fc_W[OUT:]], axis=1)
    buv = jnp.concatenate([fc_b, jnp.zeros((1,), _f32)]).reshape(1, 2)
    uv = _m3(root2, agg2_0, agg2_1, cnt0, cnt1, wuv, buv)
    out = _decode_cache()(uv[:, 0], uv[:, 1], i0, i1)
    return out.reshape(Q, 1)
